# baseline (speedup 1.0000x reference)
import functools
import jax
import jax.numpy as jnp
from jax.experimental import pallas as pl

_K = 32
_ALPHA = 0.1


def _mse_kernel(x_ref, y_ref, o_ref):
    d = x_ref[...] - y_ref[...]
    o_ref[...] = jnp.sum(d * d).reshape(1, 1)


def _topk_kernel(a_ref, bot_ref, *, n_iter):
    a = a_ref[...]
    ai = jax.lax.bitcast_convert_type(a, jnp.int32)

    r = a.shape[0]

    h = (ai >> 15).astype(jnp.int16)
    plo0 = jnp.full((r, 1), -1, dtype=jnp.int32)
    phi0 = jnp.full((r, 1), 0x7F00, dtype=jnp.int32)

    def body16(_, carry):
        lo, hi = carry
        mid = lo + ((hi - lo) >> 1)
        mid16 = mid.astype(jnp.int16)
        c = jnp.sum((h > mid16).astype(jnp.float32), axis=1, keepdims=True)
        take = c >= _K
        lo = jnp.where(take, mid, lo)
        hi = jnp.where(take, hi, mid)
        return lo, hi

    plo, phi = jax.lax.fori_loop(0, 15, body16, (plo0, phi0))
    p32 = phi
    lo0 = (p32 << 15) - 1
    hi0 = (p32 << 15) + 0x7FFF

    def body(_, carry):
        lo, hi = carry
        mid = lo + ((hi - lo) >> 1)
        c = jnp.sum((ai > mid).astype(jnp.float32), axis=1, keepdims=True)
        take = c >= _K
        lo = jnp.where(take, mid, lo)
        hi = jnp.where(take, hi, mid)
        return lo, hi

    lo, hi = jax.lax.fori_loop(0, n_iter, body, (lo0, hi0))

    t = jax.lax.bitcast_convert_type(hi, jnp.float32)
    m = ai > hi
    sq = a * a
    c_gt = jnp.sum(m.astype(jnp.float32), axis=1, keepdims=True)
    s_le = jnp.sum(jnp.where(m, 0.0, sq), axis=1, keepdims=True)
    bot = s_le - (_K - c_gt) * (t * t)
    bot_ref[...] = jnp.sum(bot).reshape(1, 1, 1)


def kernel(x, y, attn):
    s = attn.shape[-1]
    rows = attn.size // s
    a2 = attn.reshape(rows, s)

    block_r = min(512, rows)
    grid = rows // block_r

    bot = pl.pallas_call(
        functools.partial(_topk_kernel, n_iter=16),
        grid=(grid,),
        in_specs=[pl.BlockSpec((block_r, s), lambda i: (i, 0))],
        out_specs=pl.BlockSpec((1, 1, 1), lambda i: (i, 0, 0)),
        out_shape=jax.ShapeDtypeStruct((grid, 1, 1), jnp.float32),
    )(a2)

    x2 = x.reshape(-1, x.shape[-1])
    y2 = y.reshape(-1, y.shape[-1])
    sse = pl.pallas_call(
        _mse_kernel,
        out_specs=pl.BlockSpec((1, 1), lambda: (0, 0)),
        out_shape=jax.ShapeDtypeStruct((1, 1), jnp.float32),
    )(x2, y2)

    rec_loss = sse[0, 0] / x.size
    attn_loss = jnp.sum(bot) / attn.size
    return rec_loss + _ALPHA * attn_loss


# speedup vs baseline: 1.4248x; 1.4248x over previous
import functools
import jax
import jax.numpy as jnp
from jax.experimental import pallas as pl

_K = 32
_ALPHA = 0.1


def _mse_kernel(x_ref, y_ref, o_ref):
    d = x_ref[...] - y_ref[...]
    o_ref[...] = jnp.sum(d * d).reshape(1, 1)


def _topk_kernel(a_ref, bot_ref, *, n_iter, n_split):
    a = a_ref[...]
    ai = jax.lax.bitcast_convert_type(a, jnp.int32)

    r = a.shape[0]
    rs = r // n_split
    parts = [ai[i * rs:(i + 1) * rs] for i in range(n_split)]

    lo0 = jnp.full((rs, 1), -1, dtype=jnp.int32)
    hi0 = jnp.full((rs, 1), 0x7F800000, dtype=jnp.int32)

    def body(_, carry):
        out = []
        for (lo, hi), part in zip(carry, parts):
            mid = (lo + hi) >> 1
            c = jnp.sum((part > mid).astype(jnp.float32), axis=1, keepdims=True)
            take = c >= _K
            lo = jnp.where(take, mid, lo)
            hi = jnp.where(take, hi, mid)
            out.append((lo, hi))
        return tuple(out)

    carry0 = tuple((lo0, hi0) for _ in range(n_split))
    carry = jax.lax.fori_loop(0, n_iter, body, carry0)
    hi = jnp.concatenate([h for (_, h) in carry], axis=0)

    t = jax.lax.bitcast_convert_type(hi, jnp.float32)
    m = ai > hi
    sq = a * a
    c_gt = jnp.sum(m.astype(jnp.float32), axis=1, keepdims=True)
    s_le = jnp.sum(jnp.where(m, 0.0, sq), axis=1, keepdims=True)
    bot = s_le - (_K - c_gt) * (t * t)
    bot_ref[...] = jnp.sum(bot).reshape(1, 1, 1)


def kernel(x, y, attn):
    s = attn.shape[-1]
    rows = attn.size // s
    a2 = attn.reshape(rows, s)

    block_r = min(512, rows)
    grid = rows // block_r

    bot = pl.pallas_call(
        functools.partial(_topk_kernel, n_iter=31, n_split=2),
        grid=(grid,),
        in_specs=[pl.BlockSpec((block_r, s), lambda i: (i, 0))],
        out_specs=pl.BlockSpec((1, 1, 1), lambda i: (i, 0, 0)),
        out_shape=jax.ShapeDtypeStruct((grid, 1, 1), jnp.float32),
    )(a2)

    x2 = x.reshape(-1, x.shape[-1])
    y2 = y.reshape(-1, y.shape[-1])
    sse = pl.pallas_call(
        _mse_kernel,
        out_specs=pl.BlockSpec((1, 1), lambda: (0, 0)),
        out_shape=jax.ShapeDtypeStruct((1, 1), jnp.float32),
    )(x2, y2)

    rec_loss = sse[0, 0] / x.size
    attn_loss = jnp.sum(bot) / attn.size
    return rec_loss + _ALPHA * attn_loss
